# baseline (speedup 1.0000x reference)
import functools

import jax
import jax.numpy as jnp
from jax import lax
from jax.experimental import pallas as pl
from jax.experimental.pallas import tpu as pltpu
from jax.experimental.pallas import tpu_sc as plsc

_NC = 2
_NS = 16
_NW = _NC * _NS
_L = 16

_CHUNK = 256


@functools.lru_cache(maxsize=None)
def _make_lookup(B: int, V2: int, D: int):
    b_per_w = B // _NW
    n_chunks = b_per_w // _CHUNK
    assert B % _NW == 0 and b_per_w % _CHUNK == 0 and n_chunks % 2 == 0
    mesh = plsc.VectorSubcoreMesh(core_axis_name="c", subcore_axis_name="s")

    @functools.partial(
        pl.kernel,
        mesh=mesh,
        out_type=jax.ShapeDtypeStruct((B // 2, 2 * D), jnp.float32),
        scratch_types=[
            pltpu.VMEM((b_per_w,), jnp.int32),
            pltpu.VMEM((_CHUNK,), jnp.int32),
            pltpu.VMEM((_CHUNK,), jnp.int32),
            pltpu.VMEM((_CHUNK, 2 * D), jnp.float32),
            pltpu.VMEM((_CHUNK, 2 * D), jnp.float32),
            pltpu.VMEM((_CHUNK // 2, 2 * D), jnp.float32),
            pltpu.VMEM((_CHUNK // 2, 2 * D), jnp.float32),
            pltpu.SemaphoreType.DMA,
            pltpu.SemaphoreType.DMA,
            pltpu.SemaphoreType.DMA,
            pltpu.SemaphoreType.DMA,
        ],
        compiler_params=pltpu.CompilerParams(
            use_tc_tiling_on_sc=False, needs_layout_passes=False),
    )
    def lookup(idx_hbm, table2_hbm, out2_hbm, idx_v, gi0, gi1,
               g0, g1, o0, o1, gsem0, gsem1, ssem0, ssem1):
        wid = lax.axis_index("s") * _NC + lax.axis_index("c")
        base = wid * b_per_w
        pltpu.sync_copy(idx_hbm.at[pl.ds(base, b_per_w)], idx_v)

        def start_gather(c, gi, g, sem):
            off = c * _CHUNK

            @pl.loop(0, _CHUNK // _L)
            def _(j):
                iv = idx_v[pl.ds(off + j * _L, _L)]
                gi[pl.ds(j * _L, _L)] = lax.shift_right_logical(iv, 1)

            pltpu.make_async_copy(table2_hbm.at[gi], g, sem).start()

        def wait_gather(gi, g, sem):
            pltpu.make_async_copy(table2_hbm.at[gi], g, sem).wait()

        def extract(c, g, o):
            off = c * _CHUNK
            iota = lax.iota(jnp.int32, _L)

            @pl.loop(0, _CHUNK // _L)
            def _(j):
                r0 = j * _L
                iv = idx_v[pl.ds(off + r0, _L)]
                sb = lax.bitwise_and(iv, 1) * D
                srow = r0 + iota
                dflat = srow * D
                drow = lax.shift_right_logical(dflat, 7)
                dc0 = lax.bitwise_and(dflat, 127)

                for kk in range(D):
                    v = plsc.load_gather(g, [srow, sb + kk])
                    plsc.store_scatter(o, [drow, dc0 + kk], v)

        def store(c, o, sem):
            return pltpu.make_async_copy(
                o, out2_hbm.at[pl.ds((base + c * _CHUNK) // 2, _CHUNK // 2)],
                sem)

        start_gather(0, gi0, g0, gsem0)

        @pl.loop(0, n_chunks, step=2)
        def _(c):
            wait_gather(gi0, g0, gsem0)
            start_gather(c + 1, gi1, g1, gsem1)

            @pl.when(c > 0)
            def _():
                store(c - 2, o0, ssem0).wait()
            extract(c, g0, o0)
            store(c, o0, ssem0).start()

            wait_gather(gi1, g1, gsem1)

            @pl.when(c + 2 < n_chunks)
            def _():
                start_gather(c + 2, gi0, g0, gsem0)

            @pl.when(c > 0)
            def _():
                store(c - 1, o1, ssem1).wait()
            extract(c + 1, g1, o1)
            store(c + 1, o1, ssem1).start()

        store(n_chunks - 2, o0, ssem0).wait()
        store(n_chunks - 1, o1, ssem1).wait()

    return lookup


def kernel(x, table):
    B, L = x.shape
    V, D = table.shape
    flat_idx = x.reshape(B * L)
    table2 = table.reshape(V // 2, 2 * D)
    out2 = _make_lookup(B * L, V // 2, D)(flat_idx, table2)
    return out2.reshape(B, L, D)


# speedup vs baseline: 1.7800x; 1.7800x over previous
import functools

import jax
import jax.numpy as jnp
from jax import lax
from jax.experimental import pallas as pl
from jax.experimental.pallas import tpu as pltpu
from jax.experimental.pallas import tpu_sc as plsc

_NC = 2
_NS = 16
_NW = _NC * _NS

_CHUNK = 640


@functools.lru_cache(maxsize=None)
def _make_lookup(B: int, V: int, D: int):
    b_per_w = B // _NW
    n_chunks = b_per_w // _CHUNK
    assert B % _NW == 0 and b_per_w % _CHUNK == 0 and n_chunks % 2 == 0
    mesh = plsc.VectorSubcoreMesh(core_axis_name="c", subcore_axis_name="s")

    @functools.partial(
        pl.kernel,
        mesh=mesh,
        out_type=jax.ShapeDtypeStruct((B, D), jnp.float32),
        scratch_types=[
            pltpu.VMEM((b_per_w,), jnp.int32),
            pltpu.VMEM((_CHUNK, D), jnp.float32),
            pltpu.VMEM((_CHUNK, D), jnp.float32),
            pltpu.SemaphoreType.DMA,
            pltpu.SemaphoreType.DMA,
            pltpu.SemaphoreType.DMA,
            pltpu.SemaphoreType.DMA,
        ],
        compiler_params=pltpu.CompilerParams(use_tc_tiling_on_sc=False),
    )
    def lookup(idx_hbm, table_hbm, out_hbm, idx_v, rows0, rows1,
               gsem0, gsem1, ssem0, ssem1):
        wid = lax.axis_index("s") * _NC + lax.axis_index("c")
        base = wid * b_per_w
        pltpu.sync_copy(idx_hbm.at[pl.ds(base, b_per_w)], idx_v)

        def gather(c, buf, sem):
            return pltpu.make_async_copy(
                table_hbm.at[idx_v.at[pl.ds(c * _CHUNK, _CHUNK)]], buf, sem)

        def store(c, buf, sem):
            return pltpu.make_async_copy(
                buf, out_hbm.at[pl.ds(base + c * _CHUNK, _CHUNK)], sem)

        gather(0, rows0, gsem0).start()

        @pl.loop(0, n_chunks, step=2)
        def _(g):
            @pl.when(g > 0)
            def _():
                store(g - 1, rows1, ssem1).wait()
            gather(g + 1, rows1, gsem1).start()
            gather(g, rows0, gsem0).wait()
            store(g, rows0, ssem0).start()

            store(g, rows0, ssem0).wait()
            @pl.when(g + 2 < n_chunks)
            def _():
                gather(g + 2, rows0, gsem0).start()
            gather(g + 1, rows1, gsem1).wait()
            store(g + 1, rows1, ssem1).start()

        store(n_chunks - 1, rows1, ssem1).wait()

    return lookup


def kernel(x, table):
    B, L = x.shape
    V, D = table.shape
    flat_idx = x.reshape(B * L)
    out = _make_lookup(B * L, V, D)(flat_idx, table)
    return out.reshape(B, L, D)
